# Initial kernel scaffold; baseline (speedup 1.0000x reference)
import jax
import jax.numpy as jnp
from jax.experimental import pallas as pl


def kernel(hidden_states, gate_w, w1, w2, w3):
    raise NotImplementedError("write your pallas kernel here")


# speedup vs baseline: 1.1969x; 1.1969x over previous
import functools

import jax
import jax.numpy as jnp
from jax import lax
from jax.experimental import pallas as pl
from jax.experimental.pallas import tpu as pltpu
from jax.experimental.pallas import tpu_sc as plsc

_T = 2048
_H = 2048
_DFF = 1024
_E = 8
_B = 128
_NBLK = (_T * 2) // _B + _E
_NPAD = _NBLK * _B

_NC, _NS, _L = 2, 16, 16
_NW = _NC * _NS
_TW = _T // _NW
_CH = 16


def _router_body(x_ref, gw_ref, p0_ref, p1_ref, w0_ref, w1_ref, be_ref):
    x = x_ref[...]
    gw = gw_ref[...]
    logits = lax.dot_general(x, gw, (((1,), (1,)), ((), ())),
                             preferred_element_type=jnp.float32)
    ie = lax.broadcasted_iota(jnp.int32, (_T, _E), 1)
    m1 = jnp.max(logits, axis=1, keepdims=True)
    a1 = jnp.min(jnp.where(logits == m1, ie, _E), axis=1, keepdims=True)
    masked = jnp.where(ie == a1, -1e30, logits)
    m2 = jnp.max(masked, axis=1, keepdims=True)
    a2 = jnp.min(jnp.where(masked == m2, ie, _E), axis=1, keepdims=True)
    w0 = 1.0 / (1.0 + jnp.exp(m2 - m1))
    oh1 = (ie == a1).astype(jnp.float32)
    oh2 = (ie == a2).astype(jnp.float32)
    c = oh1 + oh2
    ri = lax.broadcasted_iota(jnp.int32, (_T, _T), 0)
    ci = lax.broadcasted_iota(jnp.int32, (_T, _T), 1)
    ltri = (ci < ri).astype(jnp.float32)
    cnt_excl = lax.dot_general(ltri, c, (((1,), (0,)), ((), ())),
                               preferred_element_type=jnp.float32)
    counts = jnp.sum(c, axis=0, keepdims=True)
    nblocks = jnp.floor((counts + (_B - 1)) / _B)
    re = lax.broadcasted_iota(jnp.int32, (_E, _E), 0)
    ce = lax.broadcasted_iota(jnp.int32, (_E, _E), 1)
    ut = (re < ce).astype(jnp.float32)
    blk_excl = lax.dot_general(nblocks, ut, (((1,), (0,)), ((), ())),
                               preferred_element_type=jnp.float32)
    slot = _B * blk_excl + cnt_excl
    p0_ref[...] = jnp.sum(oh1 * slot, axis=1, keepdims=True).astype(jnp.int32)
    p1_ref[...] = jnp.sum(oh2 * slot, axis=1, keepdims=True).astype(jnp.int32)
    w0_ref[...] = jnp.broadcast_to(w0, (_T, _L))
    w1_ref[...] = jnp.broadcast_to(1.0 - w0, (_T, _L))
    blk_incl = blk_excl + nblocks
    incl_bc = lax.dot_general(blk_incl, jnp.ones((1, _NBLK), jnp.float32),
                              (((0,), (0,)), ((), ())),
                              preferred_element_type=jnp.float32)
    bi = lax.broadcasted_iota(jnp.int32, (_E, _NBLK), 1)
    be = jnp.minimum(jnp.sum((bi >= incl_bc.astype(jnp.int32)).astype(jnp.int32),
                             axis=0, keepdims=True), _E - 1)
    be_ref[...] = jnp.broadcast_to(be, (8, _NBLK))


_router = pl.pallas_call(
    _router_body,
    out_shape=(
        jax.ShapeDtypeStruct((_T, 1), jnp.int32),
        jax.ShapeDtypeStruct((_T, 1), jnp.int32),
        jax.ShapeDtypeStruct((_T, _L), jnp.float32),
        jax.ShapeDtypeStruct((_T, _L), jnp.float32),
        jax.ShapeDtypeStruct((8, _NBLK), jnp.int32),
    ),
)


def _gmm_body(be_ref, xs_ref, w1_ref, w2_ref, w3_ref, y_ref):
    x = xs_ref[...]
    h1 = lax.dot_general(x, w1_ref[0], (((1,), (1,)), ((), ())),
                         preferred_element_type=jnp.float32)
    h3 = lax.dot_general(x, w3_ref[0], (((1,), (1,)), ((), ())),
                         preferred_element_type=jnp.float32)
    g = h1 * jax.nn.sigmoid(h1) * h3
    y_ref[...] = lax.dot_general(g, w2_ref[0], (((1,), (1,)), ((), ())),
                                 preferred_element_type=jnp.float32)


_gmm = pl.pallas_call(
    _gmm_body,
    grid_spec=pltpu.PrefetchScalarGridSpec(
        num_scalar_prefetch=1,
        grid=(_NBLK,),
        in_specs=[
            pl.BlockSpec((_B, _H), lambda b, be: (b, 0)),
            pl.BlockSpec((1, _DFF, _H), lambda b, be: (be[b], 0, 0)),
            pl.BlockSpec((1, _H, _DFF), lambda b, be: (be[b], 0, 0)),
            pl.BlockSpec((1, _DFF, _H), lambda b, be: (be[b], 0, 0)),
        ],
        out_specs=pl.BlockSpec((_B, _H), lambda b, be: (b, 0)),
    ),
    out_shape=jax.ShapeDtypeStruct((_NPAD, _H), jnp.float32),
)


@functools.lru_cache(maxsize=1)
def _sc_kernels():
    mesh = plsc.VectorSubcoreMesh(core_axis_name="c", subcore_axis_name="s",
                                  num_cores=_NC, num_subcores=_NS)

    @functools.partial(
        pl.kernel,
        out_type=jax.ShapeDtypeStruct((_NPAD, _H), jnp.float32),
        mesh=mesh,
        scratch_types=[
            pltpu.VMEM((_CH, _H), jnp.float32),
            pltpu.VMEM((_CH,), jnp.int32),
            pltpu.VMEM((_CH,), jnp.int32),
            pltpu.SemaphoreType.DMA,
        ],
    )
    def dispatch(x_hbm, p0_hbm, p1_hbm, xs_hbm, rows_v, i0_v, i1_v, sem):
        wid = lax.axis_index("s") * _NC + lax.axis_index("c")
        base = wid * _TW

        def chunk(i, carry):
            b = base + i * _CH
            pltpu.sync_copy(x_hbm.at[pl.ds(b, _CH)], rows_v)
            pltpu.sync_copy(p0_hbm.at[pl.ds(b, _CH)], i0_v)
            pltpu.sync_copy(p1_hbm.at[pl.ds(b, _CH)], i1_v)
            cp0 = pltpu.async_copy(rows_v, xs_hbm.at[i0_v], sem)
            cp1 = pltpu.async_copy(rows_v, xs_hbm.at[i1_v], sem)
            cp0.wait()
            cp1.wait()
            return carry

        lax.fori_loop(0, _TW // _CH, chunk, 0)

    @functools.partial(
        pl.kernel,
        out_type=jax.ShapeDtypeStruct((_T, _H), jnp.float32),
        mesh=mesh,
        scratch_types=[
            pltpu.VMEM((_CH, _H), jnp.float32),
            pltpu.VMEM((_CH, _H), jnp.float32),
            pltpu.VMEM((_CH,), jnp.int32),
            pltpu.VMEM((_CH, _L), jnp.float32),
            pltpu.VMEM((_CH, _L), jnp.float32),
            pltpu.SemaphoreType.DMA,
        ],
    )
    def combine(y_hbm, p0_hbm, p1_hbm, w0_hbm, w1_hbm, out_hbm,
                r0_v, r1_v, idx_v, w0_v, w1_v, sem):
        wid = lax.axis_index("s") * _NC + lax.axis_index("c")
        base = wid * _TW

        def chunk(i, carry):
            b = base + i * _CH
            pltpu.sync_copy(p0_hbm.at[pl.ds(b, _CH)], idx_v)
            pltpu.async_copy(y_hbm.at[idx_v], r0_v, sem).wait()
            pltpu.sync_copy(p1_hbm.at[pl.ds(b, _CH)], idx_v)
            pltpu.async_copy(y_hbm.at[idx_v], r1_v, sem).wait()
            pltpu.sync_copy(w0_hbm.at[pl.ds(b, _CH)], w0_v)
            pltpu.sync_copy(w1_hbm.at[pl.ds(b, _CH)], w1_v)

            def tok(j, c2):
                s0 = w0_v[j]
                s1 = w1_v[j]

                def vec(v, c3):
                    col = pl.ds(pl.multiple_of(v * _L, _L), _L)
                    r0_v[j, col] = s0 * r0_v[j, col] + s1 * r1_v[j, col]
                    return c3

                lax.fori_loop(0, _H // _L, vec, 0)
                return c2

            lax.fori_loop(0, _CH, tok, 0)
            pltpu.sync_copy(r0_v, out_hbm.at[pl.ds(b, _CH)])
            return carry

        lax.fori_loop(0, _TW // _CH, chunk, 0)

    return dispatch, combine


def kernel(hidden_states, gate_w, w1, w2, w3):
    x = hidden_states.reshape(_T, _H)
    p0c, p1c, w0b, w1b, be2 = _router(x, gate_w)
    p0 = p0c.reshape(_T)
    p1 = p1c.reshape(_T)
    be = be2[0]
    dispatch, combine = _sc_kernels()
    xs = dispatch(x, p0, p1)
    y = _gmm(be, xs, w1, w2, w3)
    return combine(y, p0, p1, w0b, w1b)


# speedup vs baseline: 1.8064x; 1.5092x over previous
import functools

import jax
import jax.numpy as jnp
from jax import lax
from jax.experimental import pallas as pl
from jax.experimental.pallas import tpu as pltpu
from jax.experimental.pallas import tpu_sc as plsc

_T = 2048
_H = 2048
_DFF = 1024
_E = 8
_B = 256
_NBLK = (_T * 2) // _B + _E
_NPAD = _NBLK * _B

_NC, _NS, _L = 2, 16, 16
_NW = _NC * _NS
_TW = _T // _NW
_CH = 16


def _router_body(x_ref, gw_ref, p0_ref, p1_ref, w0_ref, w1_ref, be_ref):
    x = x_ref[...]
    gw = gw_ref[...]
    logits = lax.dot_general(x, gw, (((1,), (1,)), ((), ())),
                             preferred_element_type=jnp.float32)
    ie = lax.broadcasted_iota(jnp.int32, (_T, _E), 1)
    m1 = jnp.max(logits, axis=1, keepdims=True)
    a1 = jnp.min(jnp.where(logits == m1, ie, _E), axis=1, keepdims=True)
    masked = jnp.where(ie == a1, -1e30, logits)
    m2 = jnp.max(masked, axis=1, keepdims=True)
    a2 = jnp.min(jnp.where(masked == m2, ie, _E), axis=1, keepdims=True)
    w0 = 1.0 / (1.0 + jnp.exp(m2 - m1))
    oh1 = (ie == a1).astype(jnp.float32)
    oh2 = (ie == a2).astype(jnp.float32)
    c = oh1 + oh2
    ri = lax.broadcasted_iota(jnp.int32, (_T, _T), 0)
    ci = lax.broadcasted_iota(jnp.int32, (_T, _T), 1)
    ltri = (ci < ri).astype(jnp.float32)
    cnt_excl = lax.dot_general(ltri, c, (((1,), (0,)), ((), ())),
                               preferred_element_type=jnp.float32)
    counts = jnp.sum(c, axis=0, keepdims=True)
    nblocks = jnp.floor((counts + (_B - 1)) / _B)
    re = lax.broadcasted_iota(jnp.int32, (_E, _E), 0)
    ce = lax.broadcasted_iota(jnp.int32, (_E, _E), 1)
    ut = (re < ce).astype(jnp.float32)
    blk_excl = lax.dot_general(nblocks, ut, (((1,), (0,)), ((), ())),
                               preferred_element_type=jnp.float32)
    slot = _B * blk_excl + cnt_excl
    p0_ref[...] = jnp.sum(oh1 * slot, axis=1, keepdims=True).astype(jnp.int32)
    p1_ref[...] = jnp.sum(oh2 * slot, axis=1, keepdims=True).astype(jnp.int32)
    w0_ref[...] = jnp.broadcast_to(w0, (_T, _L))
    w1_ref[...] = jnp.broadcast_to(1.0 - w0, (_T, _L))
    blk_incl = blk_excl + nblocks
    incl_bc = lax.dot_general(blk_incl, jnp.ones((1, _NBLK), jnp.float32),
                              (((0,), (0,)), ((), ())),
                              preferred_element_type=jnp.float32)
    bi = lax.broadcasted_iota(jnp.int32, (_E, _NBLK), 1)
    be = jnp.sum((bi >= incl_bc.astype(jnp.int32)).astype(jnp.int32),
                 axis=0, keepdims=True)
    be_ref[...] = jnp.broadcast_to(be, (8, _NBLK))


_router = pl.pallas_call(
    _router_body,
    out_shape=(
        jax.ShapeDtypeStruct((_T, 1), jnp.int32),
        jax.ShapeDtypeStruct((_T, 1), jnp.int32),
        jax.ShapeDtypeStruct((_T, _L), jnp.float32),
        jax.ShapeDtypeStruct((_T, _L), jnp.float32),
        jax.ShapeDtypeStruct((8, _NBLK), jnp.int32),
    ),
)


def _gmm_body(be_ref, xs_ref, w1_ref, w2_ref, w3_ref, y_ref):
    @pl.when(be_ref[pl.program_id(0)] < _E)
    def _():
        x = xs_ref[...]
        h1 = lax.dot_general(x, w1_ref[0], (((1,), (1,)), ((), ())),
                             preferred_element_type=jnp.float32)
        h3 = lax.dot_general(x, w3_ref[0], (((1,), (1,)), ((), ())),
                             preferred_element_type=jnp.float32)
        g = h1 * jax.nn.sigmoid(h1) * h3
        y_ref[...] = lax.dot_general(g, w2_ref[0], (((1,), (1,)), ((), ())),
                                     preferred_element_type=jnp.float32)


def _wmap(b, be):
    return (jnp.minimum(be[b], _E - 1), 0, 0)


_gmm = pl.pallas_call(
    _gmm_body,
    grid_spec=pltpu.PrefetchScalarGridSpec(
        num_scalar_prefetch=1,
        grid=(_NBLK,),
        in_specs=[
            pl.BlockSpec((_B, _H), lambda b, be: (b, 0)),
            pl.BlockSpec((1, _DFF, _H), _wmap),
            pl.BlockSpec((1, _H, _DFF), _wmap),
            pl.BlockSpec((1, _DFF, _H), _wmap),
        ],
        out_specs=pl.BlockSpec((_B, _H), lambda b, be: (b, 0)),
    ),
    out_shape=jax.ShapeDtypeStruct((_NPAD, _H), jnp.float32),
    compiler_params=pltpu.CompilerParams(vmem_limit_bytes=100 * 1024 * 1024),
)


@functools.lru_cache(maxsize=1)
def _sc_kernels():
    mesh = plsc.VectorSubcoreMesh(core_axis_name="c", subcore_axis_name="s",
                                  num_cores=_NC, num_subcores=_NS)

    @functools.partial(
        pl.kernel,
        out_type=jax.ShapeDtypeStruct((_NPAD, _H), jnp.float32),
        mesh=mesh,
        scratch_types=[
            pltpu.VMEM((_CH, _H), jnp.float32),
            pltpu.VMEM((_CH,), jnp.int32),
            pltpu.VMEM((_CH,), jnp.int32),
            pltpu.SemaphoreType.DMA,
        ],
    )
    def dispatch(x_hbm, p0_hbm, p1_hbm, xs_hbm, rows_v, i0_v, i1_v, sem):
        wid = lax.axis_index("s") * _NC + lax.axis_index("c")
        base = wid * _TW

        def chunk(i, carry):
            b = base + i * _CH
            pltpu.sync_copy(x_hbm.at[pl.ds(b, _CH)], rows_v)
            pltpu.sync_copy(p0_hbm.at[pl.ds(b, _CH)], i0_v)
            pltpu.sync_copy(p1_hbm.at[pl.ds(b, _CH)], i1_v)
            cp0 = pltpu.async_copy(rows_v, xs_hbm.at[i0_v], sem)
            cp1 = pltpu.async_copy(rows_v, xs_hbm.at[i1_v], sem)
            cp0.wait()
            cp1.wait()
            return carry

        lax.fori_loop(0, _TW // _CH, chunk, 0)

    @functools.partial(
        pl.kernel,
        out_type=jax.ShapeDtypeStruct((_T, _H), jnp.float32),
        mesh=mesh,
        scratch_types=[
            pltpu.VMEM((_CH, _H), jnp.float32),
            pltpu.VMEM((_CH, _H), jnp.float32),
            pltpu.VMEM((_CH,), jnp.int32),
            pltpu.VMEM((_CH, _L), jnp.float32),
            pltpu.VMEM((_CH, _L), jnp.float32),
            pltpu.SemaphoreType.DMA,
        ],
    )
    def combine(y_hbm, p0_hbm, p1_hbm, w0_hbm, w1_hbm, out_hbm,
                r0_v, r1_v, idx_v, w0_v, w1_v, sem):
        wid = lax.axis_index("s") * _NC + lax.axis_index("c")
        base = wid * _TW

        def chunk(i, carry):
            b = base + i * _CH
            pltpu.sync_copy(p0_hbm.at[pl.ds(b, _CH)], idx_v)
            pltpu.async_copy(y_hbm.at[idx_v], r0_v, sem).wait()
            pltpu.sync_copy(p1_hbm.at[pl.ds(b, _CH)], idx_v)
            pltpu.async_copy(y_hbm.at[idx_v], r1_v, sem).wait()
            pltpu.sync_copy(w0_hbm.at[pl.ds(b, _CH)], w0_v)
            pltpu.sync_copy(w1_hbm.at[pl.ds(b, _CH)], w1_v)

            def tok(j, c2):
                s0 = w0_v[j]
                s1 = w1_v[j]

                @plsc.parallel_loop(0, _H, step=_L, unroll=8)
                def vec(col0):
                    col = pl.ds(pl.multiple_of(col0, _L), _L)
                    r0_v[j, col] = s0 * r0_v[j, col] + s1 * r1_v[j, col]

                return c2

            lax.fori_loop(0, _CH, tok, 0)
            pltpu.sync_copy(r0_v, out_hbm.at[pl.ds(b, _CH)])
            return carry

        lax.fori_loop(0, _TW // _CH, chunk, 0)

    return dispatch, combine


def kernel(hidden_states, gate_w, w1, w2, w3):
    x = hidden_states.reshape(_T, _H)
    p0c, p1c, w0b, w1b, be2 = _router(x, gate_w)
    p0 = p0c.reshape(_T)
    p1 = p1c.reshape(_T)
    be = be2[0]
    dispatch, combine = _sc_kernels()
    xs = dispatch(x, p0, p1)
    y = _gmm(be, xs, w1, w2, w3)
    return combine(y, p0, p1, w0b, w1b)


# speedup vs baseline: 2.0927x; 1.1585x over previous
import functools

import jax
import jax.numpy as jnp
from jax import lax
from jax.experimental import pallas as pl
from jax.experimental.pallas import tpu as pltpu
from jax.experimental.pallas import tpu_sc as plsc

_T = 2048
_H = 2048
_DFF = 1024
_E = 8
_B = 256
_NBLK = (_T * 2) // _B + _E
_NPAD = _NBLK * _B

_WSPLIT = 4
_NC, _NS, _L = 2, 16, 16
_NW = _NC * _NS
_TW = _T // _NW
_CH = 16
_CC = 8


def _router_body(x_ref, gw_ref, p0_ref, p1_ref, w0_ref, w1_ref, be_ref,
                 rs_ref, ne_ref, rp_ref):
    x = x_ref[...]
    gw = gw_ref[...]
    logits = lax.dot_general(x, gw, (((1,), (1,)), ((), ())),
                             preferred_element_type=jnp.float32)
    ie = lax.broadcasted_iota(jnp.int32, (_T, _E), 1)
    m1 = jnp.max(logits, axis=1, keepdims=True)
    a1 = jnp.min(jnp.where(logits == m1, ie, _E), axis=1, keepdims=True)
    masked = jnp.where(ie == a1, -1e30, logits)
    m2 = jnp.max(masked, axis=1, keepdims=True)
    a2 = jnp.min(jnp.where(masked == m2, ie, _E), axis=1, keepdims=True)
    w0 = 1.0 / (1.0 + jnp.exp(m2 - m1))
    oh1 = (ie == a1).astype(jnp.float32)
    oh2 = (ie == a2).astype(jnp.float32)
    c = oh1 + oh2
    ri = lax.broadcasted_iota(jnp.int32, (_T, _T), 0)
    ci = lax.broadcasted_iota(jnp.int32, (_T, _T), 1)
    ltri = (ci < ri).astype(jnp.float32)
    cnt_excl = lax.dot_general(ltri, c, (((1,), (0,)), ((), ())),
                               preferred_element_type=jnp.float32)
    counts = jnp.sum(c, axis=0, keepdims=True)
    nblocks = jnp.floor((counts + (_B - 1)) / _B)
    re = lax.broadcasted_iota(jnp.int32, (_E, _E), 0)
    ce = lax.broadcasted_iota(jnp.int32, (_E, _E), 1)
    ut = (re < ce).astype(jnp.float32)
    blk_excl = lax.dot_general(nblocks, ut, (((1,), (0,)), ((), ())),
                               preferred_element_type=jnp.float32)
    slot = _B * blk_excl + cnt_excl
    p0c = jnp.sum(oh1 * slot, axis=1, keepdims=True)
    p1c = jnp.sum(oh2 * slot, axis=1, keepdims=True)
    ident = (ri == ci).astype(jnp.float32)
    p0_ref[...] = lax.dot_general(p0c, ident, (((0,), (0,)), ((), ())),
                                  preferred_element_type=jnp.float32
                                  ).astype(jnp.int32)
    p1_ref[...] = lax.dot_general(p1c, ident, (((0,), (0,)), ((), ())),
                                  preferred_element_type=jnp.float32
                                  ).astype(jnp.int32)
    w0_ref[...] = jnp.broadcast_to(w0, (_T, _L))
    w1_ref[...] = jnp.broadcast_to(1.0 - w0, (_T, _L))
    blk_incl = blk_excl + nblocks
    incl_bc = lax.dot_general(blk_incl, jnp.ones((1, _NBLK), jnp.float32),
                              (((0,), (0,)), ((), ())),
                              preferred_element_type=jnp.float32)
    bi = lax.broadcasted_iota(jnp.int32, (_E, _NBLK), 1)
    be = jnp.sum((bi >= incl_bc.astype(jnp.int32)).astype(jnp.int32),
                 axis=0, keepdims=True)
    be_ref[...] = jnp.broadcast_to(be, (8, _NBLK))
    ones_nb = jnp.ones((1, _NBLK), jnp.float32)
    excl_bc = lax.dot_general(blk_excl, ones_nb, (((0,), (0,)), ((), ())),
                              preferred_element_type=jnp.float32)
    cnt_bc = lax.dot_general(counts, ones_nb, (((0,), (0,)), ((), ())),
                             preferred_element_type=jnp.float32)
    ie2 = lax.broadcasted_iota(jnp.int32, (_E, _NBLK), 0)
    rs = jnp.max(((bi == excl_bc.astype(jnp.int32)) &
                  (cnt_bc > 0.5)).astype(jnp.int32), axis=0, keepdims=True)
    be_bc = jnp.broadcast_to(be, (_E, _NBLK))
    ne = jnp.min(jnp.where((ie2 > be_bc) & (cnt_bc > 0.5), ie2, _E),
                 axis=0, keepdims=True)
    rn = lax.broadcasted_iota(jnp.int32, (_NBLK, _NBLK), 0)
    cn = lax.broadcasted_iota(jnp.int32, (_NBLK, _NBLK), 1)
    le = (rn <= cn).astype(jnp.float32)
    cumrs = lax.dot_general(rs.astype(jnp.float32), le,
                            (((1,), (0,)), ((), ())),
                            preferred_element_type=jnp.float32)
    cr = cumrs - 1.0
    rp = (cr - 2.0 * jnp.floor(cr * 0.5)).astype(jnp.int32)
    rs_ref[...] = jnp.broadcast_to(rs, (8, _NBLK))
    ne_ref[...] = jnp.broadcast_to(ne, (8, _NBLK))
    rp_ref[...] = jnp.broadcast_to(rp, (8, _NBLK))


_router = pl.pallas_call(
    _router_body,
    out_shape=(
        jax.ShapeDtypeStruct((1, _T), jnp.int32),
        jax.ShapeDtypeStruct((1, _T), jnp.int32),
        jax.ShapeDtypeStruct((_T, _L), jnp.float32),
        jax.ShapeDtypeStruct((_T, _L), jnp.float32),
        jax.ShapeDtypeStruct((8, _NBLK), jnp.int32),
        jax.ShapeDtypeStruct((8, _NBLK), jnp.int32),
        jax.ShapeDtypeStruct((8, _NBLK), jnp.int32),
        jax.ShapeDtypeStruct((8, _NBLK), jnp.int32),
    ),
)


def _tailmap(b, be):
    return jnp.where(be[b] < _E, b, _NBLK - 1)


def _gmm_body(be_ref, rs_ref, ne_ref, rp_ref,
              xs_ref, w1_hbm, w2_hbm, w3_hbm, y_ref,
              w1b, w2b, w3b, sems):
    b = pl.program_id(0)
    e = be_ref[b]
    slot = rp_ref[b]

    def wcopy(eidx, s):
        cps = []
        for k, (hbm, buf, d0) in enumerate(
                ((w1_hbm, w1b, _DFF), (w2_hbm, w2b, _H), (w3_hbm, w3b, _DFF))):
            ch = d0 // _WSPLIT
            for c in range(_WSPLIT):
                cps.append(pltpu.make_async_copy(
                    hbm.at[eidx, pl.ds(c * ch, ch)],
                    buf.at[s, pl.ds(c * ch, ch)],
                    sems.at[k, c, s]))
        return cps

    @pl.when(b == 0)
    def _():
        for cp in wcopy(e, slot):
            cp.start()

    @pl.when((e < _E) & (rs_ref[b] == 1))
    def _():
        for cp in wcopy(e, slot):
            cp.wait()

        @pl.when(ne_ref[b] < _E)
        def _():
            for cp in wcopy(ne_ref[b], 1 - slot):
                cp.start()

    @pl.when(e < _E)
    def _():
        x = xs_ref[...]
        h1 = lax.dot_general(x, w1b[slot], (((1,), (1,)), ((), ())),
                             preferred_element_type=jnp.float32)
        h3 = lax.dot_general(x, w3b[slot], (((1,), (1,)), ((), ())),
                             preferred_element_type=jnp.float32)
        g = h1 * jax.nn.sigmoid(h1) * h3
        y_ref[...] = lax.dot_general(g, w2b[slot], (((1,), (1,)), ((), ())),
                                     preferred_element_type=jnp.float32)


_gmm = pl.pallas_call(
    _gmm_body,
    grid_spec=pltpu.PrefetchScalarGridSpec(
        num_scalar_prefetch=4,
        grid=(_NBLK,),
        in_specs=[
            pl.BlockSpec((_B, _H), lambda b, be, rs, ne, rp: (_tailmap(b, be), 0)),
            pl.BlockSpec(memory_space=pl.ANY),
            pl.BlockSpec(memory_space=pl.ANY),
            pl.BlockSpec(memory_space=pl.ANY),
        ],
        out_specs=pl.BlockSpec((_B, _H),
                               lambda b, be, rs, ne, rp: (_tailmap(b, be), 0)),
        scratch_shapes=[
            pltpu.VMEM((2, _DFF, _H), jnp.float32),
            pltpu.VMEM((2, _H, _DFF), jnp.float32),
            pltpu.VMEM((2, _DFF, _H), jnp.float32),
            pltpu.SemaphoreType.DMA((3, _WSPLIT, 2)),
        ],
    ),
    out_shape=jax.ShapeDtypeStruct((_NPAD, _H), jnp.float32),
    compiler_params=pltpu.CompilerParams(vmem_limit_bytes=100 * 1024 * 1024),
)


@functools.lru_cache(maxsize=1)
def _sc_kernels():
    mesh = plsc.VectorSubcoreMesh(core_axis_name="c", subcore_axis_name="s",
                                  num_cores=_NC, num_subcores=_NS)

    nch = _TW // _CH

    @functools.partial(
        pl.kernel,
        out_type=jax.ShapeDtypeStruct((_NPAD, _H), jnp.float32),
        mesh=mesh,
        scratch_types=[
            pltpu.VMEM((2, _CH, _H), jnp.float32),
            pltpu.VMEM((2, _CH), jnp.int32),
            pltpu.VMEM((2, _CH), jnp.int32),
            pltpu.SemaphoreType.DMA((2,)),
        ],
    )
    def dispatch(x_hbm, p0_hbm, p1_hbm, xs_hbm, rows_v, i0_v, i1_v, sems):
        wid = lax.axis_index("s") * _NC + lax.axis_index("c")
        base = wid * _TW

        def scat(par):
            return (
                pltpu.make_async_copy(rows_v.at[par], xs_hbm.at[i0_v.at[par]],
                                      sems.at[par]),
                pltpu.make_async_copy(rows_v.at[par], xs_hbm.at[i1_v.at[par]],
                                      sems.at[par]),
            )

        def chunk(g, carry):
            par = lax.rem(g, 2)
            b = base + g * _CH

            @pl.when(g >= 2)
            def _():
                for cp in scat(par):
                    cp.wait()

            pltpu.sync_copy(x_hbm.at[pl.ds(b, _CH)], rows_v.at[par])
            pltpu.sync_copy(p0_hbm.at[pl.ds(b, _CH)], i0_v.at[par])
            pltpu.sync_copy(p1_hbm.at[pl.ds(b, _CH)], i1_v.at[par])
            for cp in scat(par):
                cp.start()
            return carry

        lax.fori_loop(0, nch, chunk, 0)
        for par in range(2):
            for cp in scat(par):
                cp.wait()

    ncc = _TW // _CC

    @functools.partial(
        pl.kernel,
        out_type=jax.ShapeDtypeStruct((_T, _H), jnp.float32),
        mesh=mesh,
        scratch_types=[
            pltpu.VMEM((2, _CC, _H), jnp.float32),
            pltpu.VMEM((2, _CC, _H), jnp.float32),
            pltpu.VMEM((2, _CC), jnp.int32),
            pltpu.VMEM((2, _CC), jnp.int32),
            pltpu.VMEM((2, _CC, _L), jnp.float32),
            pltpu.VMEM((2, _CC, _L), jnp.float32),
            pltpu.SemaphoreType.DMA((2,)),
            pltpu.SemaphoreType.DMA((2,)),
        ],
    )
    def combine(y_hbm, p0_hbm, p1_hbm, w0_hbm, w1_hbm, out_hbm,
                r0_v, r1_v, i0_v, i1_v, w0_v, w1_v, gsem, wsem):
        wid = lax.axis_index("s") * _NC + lax.axis_index("c")
        base = wid * _TW

        def gath(par):
            return (
                pltpu.make_async_copy(y_hbm.at[i0_v.at[par]], r0_v.at[par],
                                      gsem.at[par]),
                pltpu.make_async_copy(y_hbm.at[i1_v.at[par]], r1_v.at[par],
                                      gsem.at[par]),
            )

        def owrite(g, par):
            return pltpu.make_async_copy(r0_v.at[par],
                                         out_hbm.at[pl.ds(base + g * _CC, _CC)],
                                         wsem.at[par])

        def issue(g, par):
            b = base + g * _CC
            pltpu.sync_copy(p0_hbm.at[pl.ds(b, _CC)], i0_v.at[par])
            pltpu.sync_copy(p1_hbm.at[pl.ds(b, _CC)], i1_v.at[par])
            pltpu.sync_copy(w0_hbm.at[pl.ds(b, _CC)], w0_v.at[par])
            pltpu.sync_copy(w1_hbm.at[pl.ds(b, _CC)], w1_v.at[par])
            for cp in gath(par):
                cp.start()

        issue(0, 0)

        def chunk(g, carry):
            par = lax.rem(g, 2)
            npar = 1 - par

            @pl.when(g >= 1)
            def _():
                owrite(g - 1, npar).wait()

            @pl.when(g + 1 < ncc)
            def _():
                issue(g + 1, npar)

            for cp in gath(par):
                cp.wait()

            def tok(j, c2):
                s0 = w0_v[par, j]
                s1 = w1_v[par, j]

                @plsc.parallel_loop(0, _H, step=_L, unroll=8)
                def vec(col0):
                    col = pl.ds(pl.multiple_of(col0, _L), _L)
                    r0_v[par, j, col] = (s0 * r0_v[par, j, col] +
                                        s1 * r1_v[par, j, col])

                return c2

            lax.fori_loop(0, _CC, tok, 0)
            owrite(g, par).start()
            return carry

        lax.fori_loop(0, ncc, chunk, 0)
        owrite(ncc - 1, lax.rem(ncc - 1, 2)).wait()

    return dispatch, combine


def kernel(hidden_states, gate_w, w1, w2, w3):
    x = hidden_states.reshape(_T, _H)
    p0c, p1c, w0b, w1b, be2, rs2, ne2, rp2 = _router(x, gate_w)
    p0 = p0c.reshape(_T)
    p1 = p1c.reshape(_T)
    dispatch, combine = _sc_kernels()
    xs = dispatch(x, p0, p1)
    y = _gmm(be2[0], rs2[0], ne2[0], rp2[0], xs, w1, w2, w3)
    return combine(y, p0, p1, w0b, w1b)
